# Initial kernel scaffold; baseline (speedup 1.0000x reference)
import jax
import jax.numpy as jnp
from jax.experimental import pallas as pl


def kernel(x, edge_index, W1, b1, W2, b2, prelu_a, fc_W, fc_b):
    raise NotImplementedError("write your pallas kernel here")


# speedup vs baseline: 20.6530x; 20.6530x over previous
import functools

import jax
import jax.numpy as jnp
from jax import lax
from jax.experimental import pallas as pl
from jax.experimental.pallas import tpu as pltpu
from jax.experimental.pallas import tpu_sc as plsc

N = 10000
D = 128
H1 = 512
H2 = 128
OUT = 40

N_PAD = 10240
N_TILES = 32
K = 128
CH = 81
E_PAD = N_TILES * CH * K
ROWS_PT = N_PAD // 16
BLK = 256
GRID = N_PAD // BLK


def _deg_body(dst_hbm, zeros_hbm, out_hbm, idx_v, ones_v, acc_s):
    c = lax.axis_index("c")
    s = lax.axis_index("s")
    g = c * 16 + s
    pltpu.sync_copy(zeros_hbm.at[pl.ds(s * ROWS_PT, ROWS_PT)],
                    acc_s.at[pl.ds(s * ROWS_PT, ROWS_PT)])
    for i in range(K // 16):
        ones_v[pl.ds(i * 16, 16)] = jnp.ones((16,), jnp.float32)
    pltpu.sync_copy(dst_hbm.at[g], idx_v)
    plsc.subcore_barrier()

    def body(j, carry):
        pltpu.sync_copy(ones_v, acc_s.at[idx_v.at[j]], add=True)
        return carry

    lax.fori_loop(0, CH, body, 0)
    plsc.subcore_barrier()
    pltpu.sync_copy(acc_s.at[pl.ds(s * ROWS_PT, ROWS_PT)],
                    out_hbm.at[c, pl.ds(s * ROWS_PT, ROWS_PT)])


def _agg_body(tab_hbm, src_hbm, dst_hbm, zeros_hbm, out_hbm,
              src_v, dst_v, rows_v, acc_s, sem):
    c = lax.axis_index("c")
    s = lax.axis_index("s")
    g = c * 16 + s
    pltpu.sync_copy(zeros_hbm.at[pl.ds(s * ROWS_PT, ROWS_PT)],
                    acc_s.at[pl.ds(s * ROWS_PT, ROWS_PT)])
    pltpu.sync_copy(src_hbm.at[g], src_v)
    pltpu.sync_copy(dst_hbm.at[g], dst_v)
    plsc.subcore_barrier()

    def body(j, carry):
        pltpu.async_copy(tab_hbm.at[src_v.at[j]], rows_v, sem).wait()
        pltpu.sync_copy(rows_v, acc_s.at[dst_v.at[j]], add=True)
        return carry

    lax.fori_loop(0, CH, body, 0)
    plsc.subcore_barrier()
    pltpu.sync_copy(acc_s.at[pl.ds(s * ROWS_PT, ROWS_PT)],
                    out_hbm.at[c, pl.ds(s * ROWS_PT, ROWS_PT)])


@functools.cache
def _sc_kernels():
    mesh = plsc.VectorSubcoreMesh(core_axis_name="c", subcore_axis_name="s")
    deg_kernel = pl.kernel(
        _deg_body,
        mesh=mesh,
        out_type=jax.ShapeDtypeStruct((2, N_PAD), jnp.float32),
        scratch_types=[
            pltpu.VMEM((CH, K), jnp.int32),
            pltpu.VMEM((K,), jnp.float32),
            pltpu.VMEM_SHARED((N_PAD,), jnp.float32),
        ],
    )
    agg_kernel = pl.kernel(
        _agg_body,
        mesh=mesh,
        out_type=jax.ShapeDtypeStruct((2, N_PAD, D), jnp.float32),
        scratch_types=[
            pltpu.VMEM((CH, K), jnp.int32),
            pltpu.VMEM((CH, K), jnp.int32),
            pltpu.VMEM((K, D), jnp.float32),
            pltpu.VMEM_SHARED((N_PAD, D), jnp.float32),
            pltpu.SemaphoreType.DMA,
        ],
    )
    return deg_kernel, agg_kernel


def _dis_from(deg_blk):
    d = deg_blk[0, 0] + deg_blk[1, 0]
    dis = jnp.where(d > 0, lax.rsqrt(d), 0.0)
    return dis.reshape(BLK, 1)


def _t1_body(deg_ref, x_ref, out_ref):
    out_ref[...] = x_ref[...] * _dis_from(deg_ref[...])


def _t2_body(deg_ref, acc_ref, w1_ref, b1_ref, w2_ref, a_ref, out_ref):
    disc = _dis_from(deg_ref[...])
    a0 = (acc_ref[0] + acc_ref[1]) * disc
    z1 = jnp.dot(a0, w1_ref[...], preferred_element_type=jnp.float32)
    z1 = z1 + b1_ref[...]
    a = a_ref[0, 0]
    f1 = jnp.where(z1 >= 0, z1, a * z1)
    h1 = jnp.dot(f1, w2_ref[...], preferred_element_type=jnp.float32)
    out_ref[...] = h1 * disc


def _t3_body(deg_ref, acc_ref, b2_ref, a_ref, fcw_ref, fcb_ref, out_ref):
    disc = _dis_from(deg_ref[...])
    a1 = (acc_ref[0] + acc_ref[1]) * disc
    z2 = a1 + b2_ref[...]
    a = a_ref[0, 0]
    f2 = jnp.where(z2 >= 0, z2, a * z2)
    out_ref[...] = jnp.dot(f2, fcw_ref[...],
                           preferred_element_type=jnp.float32) + fcb_ref[...]


def _deg_spec():
    return pl.BlockSpec((2, 1, 1, BLK), lambda b: (0, b, 0, 0))


def _row_spec(width):
    return pl.BlockSpec((BLK, width), lambda b: (b, 0))


def _full_spec(shape):
    return pl.BlockSpec(shape, lambda b: tuple(0 for _ in shape))


def _smem_spec():
    return pl.BlockSpec(memory_space=pltpu.SMEM)


_t1_call = pl.pallas_call(
    _t1_body,
    grid=(GRID,),
    in_specs=[_deg_spec(), _row_spec(D)],
    out_specs=_row_spec(D),
    out_shape=jax.ShapeDtypeStruct((N_PAD, D), jnp.float32),
)

_t2_call = pl.pallas_call(
    _t2_body,
    grid=(GRID,),
    in_specs=[
        _deg_spec(),
        pl.BlockSpec((2, BLK, D), lambda b: (0, b, 0)),
        _full_spec((D, H1)),
        _full_spec((1, H1)),
        _full_spec((H1, H2)),
        _smem_spec(),
    ],
    out_specs=_row_spec(H2),
    out_shape=jax.ShapeDtypeStruct((N_PAD, H2), jnp.float32),
)

_t3_call = pl.pallas_call(
    _t3_body,
    grid=(GRID,),
    in_specs=[
        _deg_spec(),
        pl.BlockSpec((2, BLK, H2), lambda b: (0, b, 0)),
        _full_spec((1, H2)),
        _smem_spec(),
        _full_spec((H2, 128)),
        _full_spec((1, 128)),
    ],
    out_specs=_row_spec(128),
    out_shape=jax.ShapeDtypeStruct((N_PAD, 128), jnp.float32),
)


@jax.jit
def kernel(x, edge_index, W1, b1, W2, b2, prelu_a, fc_W, fc_b):
    loop = jnp.arange(N, dtype=jnp.int32)
    pad = jnp.full((E_PAD - edge_index.shape[1] - N,), N, dtype=jnp.int32)
    src_r = jnp.concatenate([edge_index[0], loop, pad]).reshape(N_TILES, CH, K)
    dst_r = jnp.concatenate([edge_index[1], loop, pad]).reshape(N_TILES, CH, K)

    x_pad = jnp.zeros((N_PAD, D), jnp.float32).at[:N].set(x)
    zeros_vec = jnp.zeros((N_PAD,), jnp.float32)
    zeros_tab = jnp.zeros((N_PAD, D), jnp.float32)
    a11 = jnp.asarray(prelu_a, jnp.float32).reshape(1, 1)
    fcw_pad = jnp.zeros((H2, 128), jnp.float32).at[:, :OUT].set(fc_W)
    fcb_pad = jnp.zeros((1, 128), jnp.float32).at[0, :OUT].set(fc_b)

    _deg_kernel, _agg_kernel = _sc_kernels()
    deg = _deg_kernel(dst_r, zeros_vec)
    deg_r = deg.reshape(2, GRID, 1, BLK)

    g0 = _t1_call(deg_r, x_pad)
    acc0 = _agg_kernel(g0, src_r, dst_r, zeros_tab)
    g1 = _t2_call(deg_r, acc0, W1, b1.reshape(1, H1), W2, a11)
    acc1 = _agg_kernel(g1, src_r, dst_r, zeros_tab)
    out = _t3_call(deg_r, acc1, b2.reshape(1, H2), a11, fcw_pad, fcb_pad)
    return out[:N, :OUT]


# speedup vs baseline: 36.7355x; 1.7787x over previous
import functools

import jax
import jax.numpy as jnp
from jax import lax
from jax.experimental import pallas as pl
from jax.experimental.pallas import tpu as pltpu
from jax.experimental.pallas import tpu_sc as plsc

N = 10000
D = 128
H1 = 512
H2 = 128
OUT = 40

N_PAD = 10240
N_TILES = 32
K = 128
CH = 84
E_PAD = N_TILES * CH * K
ROWS_PT = N_PAD // 16
BLK = 256
GRID = N_PAD // BLK


def _deg_body(dst_hbm, zeros_hbm, out_hbm, idx_v, ones_v, acc_s):
    c = lax.axis_index("c")
    s = lax.axis_index("s")
    g = c * 16 + s
    pltpu.sync_copy(zeros_hbm.at[pl.ds(s * ROWS_PT, ROWS_PT)],
                    acc_s.at[pl.ds(s * ROWS_PT, ROWS_PT)])
    for i in range(K // 16):
        ones_v[pl.ds(i * 16, 16)] = jnp.ones((16,), jnp.float32)
    pltpu.sync_copy(dst_hbm.at[g], idx_v)
    plsc.subcore_barrier()

    def body(j, carry):
        pltpu.sync_copy(ones_v, acc_s.at[idx_v.at[j]], add=True)
        return carry

    lax.fori_loop(0, CH, body, 0)
    plsc.subcore_barrier()
    pltpu.sync_copy(acc_s.at[pl.ds(s * ROWS_PT, ROWS_PT)],
                    out_hbm.at[c, pl.ds(s * ROWS_PT, ROWS_PT)])


def _agg_body(tab_hbm, sd_hbm, zeros_hbm, out_hbm,
              sd_bufs, row_bufs, acc_s, sem_i, sem_g):
    c = lax.axis_index("c")
    s = lax.axis_index("s")
    g = c * 16 + s
    pltpu.sync_copy(zeros_hbm.at[pl.ds(s * ROWS_PT, ROWS_PT)],
                    acc_s.at[pl.ds(s * ROWS_PT, ROWS_PT)])

    for b in range(3):
        pltpu.async_copy(sd_hbm.at[g, b], sd_bufs[b], sem_i[b])
    pltpu.make_async_copy(sd_hbm.at[g, 0], sd_bufs[0], sem_i[0]).wait()
    pltpu.async_copy(tab_hbm.at[sd_bufs[0].at[0]], row_bufs[0], sem_g[0])
    pltpu.async_copy(sd_hbm.at[g, 3], sd_bufs[3], sem_i[3])
    plsc.subcore_barrier()

    def body(i, carry):
        j0 = 4 * i
        for b in range(4):
            j = j0 + b
            bn = (b + 1) % 4
            rn = (b + 1) % 2

            @pl.when(j + 1 < CH)
            def _():
                pltpu.make_async_copy(sd_hbm.at[g, j + 1],
                                      sd_bufs[bn], sem_i[bn]).wait()
                pltpu.async_copy(tab_hbm.at[sd_bufs[bn].at[0]],
                                 row_bufs[rn], sem_g[rn])

            pltpu.make_async_copy(tab_hbm.at[sd_bufs[b].at[0]],
                                  row_bufs[b % 2], sem_g[b % 2]).wait()
            pltpu.sync_copy(row_bufs[b % 2], acc_s.at[sd_bufs[b].at[1]],
                            add=True)

            @pl.when(j + 4 < CH)
            def _():
                pltpu.async_copy(sd_hbm.at[g, j + 4], sd_bufs[b], sem_i[b])

        return carry

    lax.fori_loop(0, CH // 4, body, 0)
    plsc.subcore_barrier()
    pltpu.sync_copy(acc_s.at[pl.ds(s * ROWS_PT, ROWS_PT)],
                    out_hbm.at[c, pl.ds(s * ROWS_PT, ROWS_PT)])


@functools.cache
def _sc_kernels():
    mesh = plsc.VectorSubcoreMesh(core_axis_name="c", subcore_axis_name="s")
    deg_kernel = pl.kernel(
        _deg_body,
        mesh=mesh,
        out_type=jax.ShapeDtypeStruct((2, N_PAD), jnp.float32),
        scratch_types=[
            pltpu.VMEM((CH, K), jnp.int32),
            pltpu.VMEM((K,), jnp.float32),
            pltpu.VMEM_SHARED((N_PAD,), jnp.float32),
        ],
    )
    agg_kernel = pl.kernel(
        _agg_body,
        mesh=mesh,
        out_type=jax.ShapeDtypeStruct((2, N_PAD, D), jnp.float32),
        scratch_types=[
            tuple(pltpu.VMEM((2, K), jnp.int32) for _ in range(4)),
            tuple(pltpu.VMEM((K, D), jnp.float32) for _ in range(2)),
            pltpu.VMEM_SHARED((N_PAD, D), jnp.float32),
            tuple(pltpu.SemaphoreType.DMA for _ in range(4)),
            tuple(pltpu.SemaphoreType.DMA for _ in range(2)),
        ],
    )
    return deg_kernel, agg_kernel


def _dis_from(deg_blk):
    d = deg_blk[0, 0] + deg_blk[1, 0]
    dis = jnp.where(d > 0, lax.rsqrt(d), 0.0)
    return dis.reshape(BLK, 1)


def _t1_body(deg_ref, x_ref, out_ref):
    out_ref[...] = x_ref[...] * _dis_from(deg_ref[...])


def _t2_body(deg_ref, acc_ref, w1_ref, b1_ref, w2_ref, a_ref, out_ref):
    disc = _dis_from(deg_ref[...])
    a0 = (acc_ref[0] + acc_ref[1]) * disc
    z1 = jnp.dot(a0, w1_ref[...], preferred_element_type=jnp.float32)
    z1 = z1 + b1_ref[...]
    a = a_ref[0, 0]
    f1 = jnp.where(z1 >= 0, z1, a * z1)
    h1 = jnp.dot(f1, w2_ref[...], preferred_element_type=jnp.float32)
    out_ref[...] = h1 * disc


def _t3_body(deg_ref, acc_ref, b2_ref, a_ref, fcw_ref, fcb_ref, out_ref):
    disc = _dis_from(deg_ref[...])
    a1 = (acc_ref[0] + acc_ref[1]) * disc
    z2 = a1 + b2_ref[...]
    a = a_ref[0, 0]
    f2 = jnp.where(z2 >= 0, z2, a * z2)
    out_ref[...] = jnp.dot(f2, fcw_ref[...],
                           preferred_element_type=jnp.float32) + fcb_ref[...]


def _deg_spec():
    return pl.BlockSpec((2, 1, 1, BLK), lambda b: (0, b, 0, 0))


def _row_spec(width):
    return pl.BlockSpec((BLK, width), lambda b: (b, 0))


def _full_spec(shape):
    return pl.BlockSpec(shape, lambda b: tuple(0 for _ in shape))


def _smem_spec():
    return pl.BlockSpec(memory_space=pltpu.SMEM)


_t1_call = pl.pallas_call(
    _t1_body,
    grid=(GRID,),
    in_specs=[_deg_spec(), _row_spec(D)],
    out_specs=_row_spec(D),
    out_shape=jax.ShapeDtypeStruct((N_PAD, D), jnp.float32),
)

_t2_call = pl.pallas_call(
    _t2_body,
    grid=(GRID,),
    in_specs=[
        _deg_spec(),
        pl.BlockSpec((2, BLK, D), lambda b: (0, b, 0)),
        _full_spec((D, H1)),
        _full_spec((1, H1)),
        _full_spec((H1, H2)),
        _smem_spec(),
    ],
    out_specs=_row_spec(H2),
    out_shape=jax.ShapeDtypeStruct((N_PAD, H2), jnp.float32),
)

_t3_call = pl.pallas_call(
    _t3_body,
    grid=(GRID,),
    in_specs=[
        _deg_spec(),
        pl.BlockSpec((2, BLK, H2), lambda b: (0, b, 0)),
        _full_spec((1, H2)),
        _smem_spec(),
        _full_spec((H2, 128)),
        _full_spec((1, 128)),
    ],
    out_specs=_row_spec(128),
    out_shape=jax.ShapeDtypeStruct((N_PAD, 128), jnp.float32),
)


@jax.jit
def kernel(x, edge_index, W1, b1, W2, b2, prelu_a, fc_W, fc_b):
    loop = jnp.arange(N, dtype=jnp.int32)
    n_pad_e = E_PAD - edge_index.shape[1] - N
    pad = N + jnp.arange(n_pad_e, dtype=jnp.int32) % (N_PAD - N)
    src_r = jnp.concatenate([edge_index[0], loop, pad]).reshape(N_TILES, CH, K)
    dst_r = jnp.concatenate([edge_index[1], loop, pad]).reshape(N_TILES, CH, K)
    sd_r = jnp.stack([src_r, dst_r], axis=2)

    x_pad = jnp.zeros((N_PAD, D), jnp.float32).at[:N].set(x)
    zeros_vec = jnp.zeros((N_PAD,), jnp.float32)
    zeros_tab = jnp.zeros((N_PAD, D), jnp.float32)
    a11 = jnp.asarray(prelu_a, jnp.float32).reshape(1, 1)
    fcw_pad = jnp.zeros((H2, 128), jnp.float32).at[:, :OUT].set(fc_W)
    fcb_pad = jnp.zeros((1, 128), jnp.float32).at[0, :OUT].set(fc_b)

    _deg_kernel, _agg_kernel = _sc_kernels()
    deg = _deg_kernel(dst_r, zeros_vec)
    deg_r = deg.reshape(2, GRID, 1, BLK)

    g0 = _t1_call(deg_r, x_pad)
    acc0 = _agg_kernel(g0, sd_r, zeros_tab)
    g1 = _t2_call(deg_r, acc0, W1, b1.reshape(1, H1), W2, a11)
    acc1 = _agg_kernel(g1, sd_r, zeros_tab)
    out = _t3_call(deg_r, acc1, b2.reshape(1, H2), a11, fcw_pad, fcb_pad)
    return out[:N, :OUT]


# speedup vs baseline: 40.0032x; 1.0890x over previous
import functools

import jax
import jax.numpy as jnp
from jax import lax
from jax.experimental import pallas as pl
from jax.experimental.pallas import tpu as pltpu
from jax.experimental.pallas import tpu_sc as plsc

N = 10000
D = 128
H1 = 512
H2 = 128
OUT = 40

N_PAD = 10240
N_TILES = 32
K = 128
CH = 84
E_PAD = N_TILES * CH * K
ROWS_PT = N_PAD // 16
BLK = 512
GRID = N_PAD // BLK


def _deg_body(dst_hbm, zeros_hbm, out_hbm, idx_v, ones_v, acc_s):
    c = lax.axis_index("c")
    s = lax.axis_index("s")
    g = c * 16 + s
    pltpu.sync_copy(zeros_hbm.at[pl.ds(s * ROWS_PT, ROWS_PT)],
                    acc_s.at[pl.ds(s * ROWS_PT, ROWS_PT)])
    for i in range(K // 16):
        ones_v[pl.ds(i * 16, 16)] = jnp.ones((16,), jnp.float32)
    pltpu.sync_copy(dst_hbm.at[g], idx_v)
    plsc.subcore_barrier()

    def body(j, carry):
        pltpu.sync_copy(ones_v, acc_s.at[idx_v.at[j]], add=True)
        return carry

    lax.fori_loop(0, CH, body, 0)
    plsc.subcore_barrier()
    pltpu.sync_copy(acc_s.at[pl.ds(s * ROWS_PT, ROWS_PT)],
                    out_hbm.at[c, pl.ds(s * ROWS_PT, ROWS_PT)])


def _agg_body(tab_hbm, sd_hbm, zeros_hbm, out_hbm,
              sd_bufs, row_bufs, acc_s, sem_i, sem_g, sem_s):
    c = lax.axis_index("c")
    s = lax.axis_index("s")
    g = c * 16 + s
    pltpu.sync_copy(zeros_hbm.at[pl.ds(s * ROWS_PT, ROWS_PT)],
                    acc_s.at[pl.ds(s * ROWS_PT, ROWS_PT)])

    for b in range(3):
        pltpu.async_copy(sd_hbm.at[g, b], sd_bufs[b], sem_i[b])
    pltpu.make_async_copy(sd_hbm.at[g, 0], sd_bufs[0], sem_i[0]).wait()
    pltpu.async_copy(tab_hbm.at[sd_bufs[0].at[0]], row_bufs[0], sem_g[0])
    pltpu.async_copy(sd_hbm.at[g, 3], sd_bufs[3], sem_i[3])
    plsc.subcore_barrier()

    def body(i, carry):
        j0 = 4 * i
        for b in range(4):
            j = j0 + b
            bn = (b + 1) % 4
            bp = (b + 3) % 4
            rn = (b + 1) % 2

            @pl.when(j >= 1)
            def _():
                pltpu.make_async_copy(tab_hbm.at[pl.ds(0, K)], row_bufs[rn],
                                      sem_s[rn]).wait()

                @pl.when(j + 3 < CH)
                def _():
                    pltpu.async_copy(sd_hbm.at[g, j + 3], sd_bufs[bp],
                                     sem_i[bp])

            @pl.when(j + 1 < CH)
            def _():
                pltpu.make_async_copy(sd_hbm.at[g, j + 1],
                                      sd_bufs[bn], sem_i[bn]).wait()
                pltpu.async_copy(tab_hbm.at[sd_bufs[bn].at[0]],
                                 row_bufs[rn], sem_g[rn])

            pltpu.make_async_copy(tab_hbm.at[sd_bufs[b].at[0]],
                                  row_bufs[b % 2], sem_g[b % 2]).wait()
            pltpu.async_copy(row_bufs[b % 2], acc_s.at[sd_bufs[b].at[1]],
                             sem_s[b % 2], add=True)

        return carry

    lax.fori_loop(0, CH // 4, body, 0)
    pltpu.make_async_copy(tab_hbm.at[pl.ds(0, K)], row_bufs[1],
                          sem_s[1]).wait()
    plsc.subcore_barrier()
    pltpu.sync_copy(acc_s.at[pl.ds(s * ROWS_PT, ROWS_PT)],
                    out_hbm.at[c, pl.ds(s * ROWS_PT, ROWS_PT)])


@functools.cache
def _sc_kernels():
    mesh = plsc.VectorSubcoreMesh(core_axis_name="c", subcore_axis_name="s")
    deg_kernel = pl.kernel(
        _deg_body,
        mesh=mesh,
        out_type=jax.ShapeDtypeStruct((2, N_PAD), jnp.float32),
        scratch_types=[
            pltpu.VMEM((CH, K), jnp.int32),
            pltpu.VMEM((K,), jnp.float32),
            pltpu.VMEM_SHARED((N_PAD,), jnp.float32),
        ],
    )
    agg_kernel = pl.kernel(
        _agg_body,
        mesh=mesh,
        out_type=jax.ShapeDtypeStruct((2, N_PAD, D), jnp.float32),
        scratch_types=[
            tuple(pltpu.VMEM((2, K), jnp.int32) for _ in range(4)),
            tuple(pltpu.VMEM((K, D), jnp.float32) for _ in range(2)),
            pltpu.VMEM_SHARED((N_PAD, D), jnp.float32),
            tuple(pltpu.SemaphoreType.DMA for _ in range(4)),
            tuple(pltpu.SemaphoreType.DMA for _ in range(2)),
            tuple(pltpu.SemaphoreType.DMA for _ in range(2)),
        ],
    )
    return deg_kernel, agg_kernel


def _t1_body(deg_ref, x_ref, out_ref, dis_ref):
    d = deg_ref[0, 0] + deg_ref[1, 0]
    dis = jnp.where(d > 0, lax.rsqrt(d), 0.0)
    disc = dis.reshape(BLK, 1)
    dis_ref[...] = disc
    out_ref[...] = x_ref[...] * disc


def _t2_body(dis_ref, acc_ref, w1_ref, b1_ref, w2_ref, a_ref, out_ref):
    disc = dis_ref[...]
    a0 = (acc_ref[0] + acc_ref[1]) * disc
    z1 = jnp.dot(a0, w1_ref[...], preferred_element_type=jnp.float32)
    z1 = z1 + b1_ref[...]
    a = a_ref[0, 0]
    f1 = jnp.where(z1 >= 0, z1, a * z1)
    h1 = jnp.dot(f1, w2_ref[...], preferred_element_type=jnp.float32)
    out_ref[...] = h1 * disc


def _t3_body(dis_ref, acc_ref, b2_ref, a_ref, fcw_ref, fcb_ref, out_ref):
    disc = dis_ref[...]
    a1 = (acc_ref[0] + acc_ref[1]) * disc
    z2 = a1 + b2_ref[...]
    a = a_ref[0, 0]
    f2 = jnp.where(z2 >= 0, z2, a * z2)
    out_ref[...] = jnp.dot(f2, fcw_ref[...],
                           preferred_element_type=jnp.float32) + fcb_ref[...]


def _deg_spec():
    return pl.BlockSpec((2, 1, 1, BLK), lambda b: (0, b, 0, 0))


def _row_spec(width):
    return pl.BlockSpec((BLK, width), lambda b: (b, 0))


def _full_spec(shape):
    return pl.BlockSpec(shape, lambda b: tuple(0 for _ in shape))


def _smem_spec():
    return pl.BlockSpec(memory_space=pltpu.SMEM)


_t1_call = pl.pallas_call(
    _t1_body,
    grid=(GRID,),
    in_specs=[_deg_spec(), _row_spec(D)],
    out_specs=[_row_spec(D), _row_spec(1)],
    out_shape=[
        jax.ShapeDtypeStruct((N_PAD, D), jnp.float32),
        jax.ShapeDtypeStruct((N_PAD, 1), jnp.float32),
    ],
)

_t2_call = pl.pallas_call(
    _t2_body,
    grid=(GRID,),
    in_specs=[
        _row_spec(1),
        pl.BlockSpec((2, BLK, D), lambda b: (0, b, 0)),
        _full_spec((D, H1)),
        _full_spec((1, H1)),
        _full_spec((H1, H2)),
        _smem_spec(),
    ],
    out_specs=_row_spec(H2),
    out_shape=jax.ShapeDtypeStruct((N_PAD, H2), jnp.float32),
)

_t3_call = pl.pallas_call(
    _t3_body,
    grid=(GRID,),
    in_specs=[
        _row_spec(1),
        pl.BlockSpec((2, BLK, H2), lambda b: (0, b, 0)),
        _full_spec((1, H2)),
        _smem_spec(),
        _full_spec((H2, 128)),
        _full_spec((1, 128)),
    ],
    out_specs=_row_spec(128),
    out_shape=jax.ShapeDtypeStruct((N_PAD, 128), jnp.float32),
)


@jax.jit
def kernel(x, edge_index, W1, b1, W2, b2, prelu_a, fc_W, fc_b):
    loop = jnp.arange(N, dtype=jnp.int32)
    n_pad_e = E_PAD - edge_index.shape[1] - N
    pad = N + jnp.arange(n_pad_e, dtype=jnp.int32) % (N_PAD - N)
    src_r = jnp.concatenate([edge_index[0], loop, pad]).reshape(N_TILES, CH, K)
    dst_r = jnp.concatenate([edge_index[1], loop, pad]).reshape(N_TILES, CH, K)
    sd_r = jnp.stack([src_r, dst_r], axis=2)

    x_pad = jnp.zeros((N_PAD, D), jnp.float32).at[:N].set(x)
    zeros_vec = jnp.zeros((N_PAD,), jnp.float32)
    zeros_tab = jnp.zeros((N_PAD, D), jnp.float32)
    a11 = jnp.asarray(prelu_a, jnp.float32).reshape(1, 1)
    fcw_pad = jnp.zeros((H2, 128), jnp.float32).at[:, :OUT].set(fc_W)
    fcb_pad = jnp.zeros((1, 128), jnp.float32).at[0, :OUT].set(fc_b)

    _deg_kernel, _agg_kernel = _sc_kernels()
    deg = _deg_kernel(dst_r, zeros_vec)
    deg_r = deg.reshape(2, GRID, 1, BLK)

    g0, dis_col = _t1_call(deg_r, x_pad)
    acc0 = _agg_kernel(g0, sd_r, zeros_tab)
    g1 = _t2_call(dis_col, acc0, W1, b1.reshape(1, H1), W2, a11)
    acc1 = _agg_kernel(g1, sd_r, zeros_tab)
    out = _t3_call(dis_col, acc1, b2.reshape(1, H2), a11, fcw_pad, fcb_pad)
    return out[:N, :OUT]
